# Initial kernel scaffold; baseline (speedup 1.0000x reference)
import jax
import jax.numpy as jnp
from jax.experimental import pallas as pl


def kernel(x, edge_index, W1a, b1a, W1b, b1b, W2a, b2a, W2b, b2b):
    raise NotImplementedError("write your pallas kernel here")


# speedup vs baseline: 4.8957x; 4.8957x over previous
import functools

import jax
import jax.numpy as jnp
from jax import lax
from jax.experimental import pallas as pl
from jax.experimental.pallas import tpu as pltpu
from jax.experimental.pallas import tpu_sc as plsc

D = 128
CB = 128
NW = 32
N_SUB = 16


def _sc_aggregate(h, zeros_pad, src_t, dst_t, ch, npad):
    rows_per_tile = npad // N_SUB
    mesh = plsc.VectorSubcoreMesh(core_axis_name="c", subcore_axis_name="s")

    @functools.partial(
        pl.kernel,
        out_type=jax.ShapeDtypeStruct((2, npad, D), jnp.float32),
        mesh=mesh,
        scratch_types=[
            pltpu.VMEM((ch, CB), jnp.int32),
            pltpu.VMEM((ch, CB), jnp.int32),
            pltpu.VMEM((CB, D), jnp.float32),
            pltpu.VMEM_SHARED((npad, D), jnp.float32),
            pltpu.SemaphoreType.DMA,
        ],
    )
    def agg(h_hbm, z_hbm, src_hbm, dst_hbm, out_hbm,
            src_v, dst_v, rows_v, acc, sem):
        cid = lax.axis_index("c")
        sid = lax.axis_index("s")
        wid = cid * N_SUB + sid
        r0 = sid * rows_per_tile
        pltpu.sync_copy(z_hbm.at[pl.ds(r0, rows_per_tile)],
                        acc.at[pl.ds(r0, rows_per_tile)])
        pltpu.sync_copy(src_hbm.at[wid], src_v)
        pltpu.sync_copy(dst_hbm.at[wid], dst_v)
        plsc.subcore_barrier()

        def body(j, carry):
            pltpu.async_copy(h_hbm.at[src_v.at[j]], rows_v, sem).wait()
            pltpu.sync_copy(rows_v, acc.at[dst_v.at[j]], add=True)
            return carry

        lax.fori_loop(0, ch, body, 0)
        plsc.subcore_barrier()
        pltpu.sync_copy(acc.at[pl.ds(r0, rows_per_tile)],
                        out_hbm.at[cid, pl.ds(r0, rows_per_tile)])

    return agg(h, zeros_pad, src_t, dst_t)


def _mlp_call(partials, h, Wa, ba, Wb, bb, final_relu):
    n = h.shape[0]
    br = 1000
    grid = (n // br,)

    def body(p_ref, h_ref, wa_ref, ba_ref, wb_ref, bb_ref, o_ref):
        a = h_ref[...] + p_ref[0] + p_ref[1]
        t = jnp.dot(a, wa_ref[...], preferred_element_type=jnp.float32)
        t = jnp.maximum(t + ba_ref[...], 0.0)
        t = jnp.dot(t, wb_ref[...], preferred_element_type=jnp.float32)
        t = t + bb_ref[...]
        if final_relu:
            t = jnp.maximum(t, 0.0)
        o_ref[...] = t

    return pl.pallas_call(
        body,
        grid=grid,
        in_specs=[
            pl.BlockSpec((2, br, D), lambda i: (0, i, 0)),
            pl.BlockSpec((br, D), lambda i: (i, 0)),
            pl.BlockSpec((D, D), lambda i: (0, 0)),
            pl.BlockSpec((1, D), lambda i: (0, 0)),
            pl.BlockSpec((D, D), lambda i: (0, 0)),
            pl.BlockSpec((1, D), lambda i: (0, 0)),
        ],
        out_specs=pl.BlockSpec((br, D), lambda i: (i, 0)),
        out_shape=jax.ShapeDtypeStruct((n, D), jnp.float32),
    )(partials, h, Wa, ba.reshape(1, D), Wb, bb.reshape(1, D))


def kernel(x, edge_index, W1a, b1a, W1b, b1b, W2a, b2a, W2b, b2b):
    n = x.shape[0]
    npad = ((n + 127) // 128) * 128 + 128 if n % 128 == 0 else -(-n // 128) * 128
    src = edge_index[0].astype(jnp.int32)
    dst = edge_index[1].astype(jnp.int32)
    e = src.shape[0]
    per_tile = -(-e // NW)
    ch = -(-per_tile // CB)
    e_pad = NW * ch * CB
    src_p = jnp.concatenate(
        [src, jnp.zeros((e_pad - e,), jnp.int32)]).reshape(NW, ch, CB)
    dst_p = jnp.concatenate(
        [dst, jnp.full((e_pad - e,), n, jnp.int32)]).reshape(NW, ch, CB)
    zeros_pad = jnp.zeros((npad, D), jnp.float32)

    p1 = _sc_aggregate(x, zeros_pad, src_p, dst_p, ch, npad)
    h1 = _mlp_call(p1, x, W1a, b1a, W1b, b1b, final_relu=True)
    p2 = _sc_aggregate(h1, zeros_pad, src_p, dst_p, ch, npad)
    out = _mlp_call(p2, h1, W2a, b2a, W2b, b2b, final_relu=False)
    return out
